# baseline (speedup 1.0000x reference)
import math

import jax
import jax.numpy as jnp
from jax import lax
from jax.experimental import pallas as pl
from jax.experimental.pallas import tpu as pltpu
from jax.experimental.pallas import tpu_sc as plsc

_NC = 2
_NS = 16
_NW = _NC * _NS
_LANES = 16


def _sc_partial_sums(flat_hbm, base_idx, n_elems):
    chunk = n_elems // _NW
    steps = chunk // (2 * 64)

    mesh = plsc.VectorSubcoreMesh(core_axis_name="c", subcore_axis_name="s")

    def body(x_hbm, base_hbm, out_hbm, base_v, chunk_v, stage_v):
        c = lax.axis_index("c")
        s = lax.axis_index("s")
        wid = s * _NC + c
        pltpu.sync_copy(base_hbm, base_v)
        pltpu.sync_copy(x_hbm.at[pl.ds(wid * chunk, chunk)], chunk_v)

        idx0 = base_v[...]
        acc0 = jnp.zeros((_LANES,), jnp.float32)

        def step(_, carry):
            idx, acc = carry
            g = plsc.load_gather(chunk_v, [idx])
            return (idx + 128, acc + g)

        _, acc = lax.fori_loop(0, steps, step, (idx0, acc0))
        stage_v[...] = acc
        pltpu.sync_copy(stage_v, out_hbm.at[wid])

    run = pl.kernel(
        body,
        mesh=mesh,
        out_type=jax.ShapeDtypeStruct((_NW, _LANES), jnp.float32),
        scratch_types=[
            pltpu.VMEM((_LANES,), jnp.int32),
            pltpu.VMEM((chunk,), jnp.float32),
            pltpu.VMEM((_LANES,), jnp.float32),
        ],
        compiler_params=pltpu.CompilerParams(needs_layout_passes=False),
    )
    return run(flat_hbm, base_idx)


def _tc_finish(partials, log_uniform, inv_scale):
    def body(p_ref, o_ref):
        total = log_uniform + inv_scale * jnp.sum(p_ref[...])
        o_ref[...] = jnp.full((1, 1), 0.0, jnp.float32) + total

    return pl.pallas_call(
        body,
        out_shape=jax.ShapeDtypeStruct((1, 1), jnp.float32),
    )(partials)


def kernel(log_probs, top_k_indices, min_k_expert_indices, layer_idx):
    b, t, e = log_probs.shape
    n = b * t
    k = min_k_expert_indices.shape[0]

    flat = log_probs.reshape(n * e)
    mink = min_k_expert_indices.astype(jnp.int32)
    base = jnp.concatenate([mink, mink + e])

    partials = _sc_partial_sums(flat, base, n * e)
    out = _tc_finish(partials, math.log(1.0 / k), -1.0 / (k * n))
    return out[0, 0]
